# baseline (speedup 1.0000x reference)
import jax
import jax.numpy as jnp
from jax import lax
from jax.experimental import pallas as pl
from jax.experimental.pallas import tpu as pltpu
from jax.experimental.pallas import tpu_sc as plsc

_N = 10000
_E = 320000
_IN = 128
_H = 8
_OUT = 16
_D = _H * _OUT
_HID = 128
_CHUNK = 128
_R = _E // _CHUNK
_NC = 2
_NS = 16
_SUB_BASE = 624
_SUB_CHUNKS = ((0, 128), (128, 128), (256, 128), (384, 128), (512, 112))
_TAIL_OFF = _SUB_BASE * _NS
_TAIL = _N - _TAIL_OFF
_LEAK = 0.2
_BLK = 2000
_GRID = _N // _BLK
_NB_BASE = _R // _NS
_NB_EXTRA = _R - _NB_BASE * _NS
_NPAIR = (_NB_BASE + _NB_EXTRA + 1) // 2


def _pre_body(x_ref, w_ref, ml_ref, mr_ref,
              feat_ref, tl_ref, tr_ref, c_ref, acc_ref):
    i = pl.program_id(1)
    x = x_ref[...]
    f = jnp.dot(x, w_ref[0], preferred_element_type=jnp.float32)
    feat_ref[0] = f
    tl = jnp.dot(f, ml_ref[0], preferred_element_type=jnp.float32)
    tr = jnp.dot(f, mr_ref[0], preferred_element_type=jnp.float32)
    tl_ref[0] = tl
    tr_ref[0] = tr
    for row, t in enumerate((tl, tr)):
        m = jnp.max(t, axis=0)
        prev = acc_ref[row, :]
        acc_ref[row, :] = jnp.where(i == 0, m, jnp.maximum(prev, m))
    zero = jnp.zeros((16,), jnp.float32)
    c_ref[0, 0, :] = jnp.maximum(zero, acc_ref[0, :] + acc_ref[1, :])


def _pre(x, w, ml, mr):
    blk = _BLK
    return pl.pallas_call(
        _pre_body,
        grid=(_NC, _GRID),
        in_specs=[
            pl.BlockSpec((blk, _IN), lambda p, i: (i, 0)),
            pl.BlockSpec((1, _IN, _D), lambda p, i: (p, 0, 0)),
            pl.BlockSpec((1, _D, 16), lambda p, i: (p, 0, 0)),
            pl.BlockSpec((1, _D, 16), lambda p, i: (p, 0, 0)),
        ],
        out_specs=[
            pl.BlockSpec((1, blk, _D), lambda p, i: (p, i, 0)),
            pl.BlockSpec((1, blk, 16), lambda p, i: (p, i, 0)),
            pl.BlockSpec((1, blk, 16), lambda p, i: (p, i, 0)),
            pl.BlockSpec((1, 1, 16), lambda p, i: (p, 0, 0)),
        ],
        out_shape=[
            jax.ShapeDtypeStruct((_NC, _N, _D), jnp.float32),
            jax.ShapeDtypeStruct((_NC, _N, 16), jnp.float32),
            jax.ShapeDtypeStruct((_NC, _N, 16), jnp.float32),
            jax.ShapeDtypeStruct((_NC, 1, 16), jnp.float32),
        ],
        scratch_shapes=[pltpu.VMEM((2, 16), jnp.float32)],
    )(x, w, ml, mr)


def _zero_shared(zbuf, sh, sid, width):
    base = pl.multiple_of(sid * _SUB_BASE, 8)
    for off, sz in _SUB_CHUNKS:
        pltpu.sync_copy(zbuf.at[pl.ds(0, sz)],
                        sh.at[pl.ds(pl.multiple_of(base + off, 8), sz)])

    @pl.when(sid == _NS - 1)
    def _zt():
        pltpu.sync_copy(zbuf.at[pl.ds(0, _TAIL)], sh.at[pl.ds(_TAIL_OFF, _TAIL)])


def _export_shared(sh, out2d_at_cid, sid):
    base = pl.multiple_of(sid * _SUB_BASE, 8)
    for off, sz in _SUB_CHUNKS:
        o = pl.multiple_of(base + off, 8)
        pltpu.sync_copy(sh.at[pl.ds(o, sz)], out2d_at_cid(o, sz))

    @pl.when(sid == _NS - 1)
    def _xt():
        pltpu.sync_copy(sh.at[pl.ds(_TAIL_OFF, _TAIL)],
                        out2d_at_cid(_TAIL_OFF, _TAIL))


def _p1_body(src_ref, dst_ref, tl_ref, tr_ref, c_ref,
             ee_ref, dp_ref,
             sx0, dx0, dg0, sx1, dx1, dg1,
             ab0, bb0, ab1, bb1, eb0, eb1,
             cbuf, dsh, sa0, sb0, sa1, sb1):
    cid = lax.axis_index("c")
    sid = lax.axis_index("s")

    def _z(i, _):
        eb0[i, :] = jnp.zeros((16,), jnp.float32)
        return 0
    lax.fori_loop(0, _CHUNK, _z, 0)
    _zero_shared(eb0, dsh, sid, 16)
    plsc.subcore_barrier()

    pltpu.sync_copy(c_ref, cbuf)
    cvec = cbuf[cid, :]
    noff = cid * _N

    nb = _NB_BASE + jnp.where(sid < _NB_EXTRA, 1, 0)

    def _issue(ci, sx, dx, dg, ab, bb, sa, sb):
        r = cid * _R + sid + _NS * ci
        pltpu.sync_copy(src_ref.at[r], sx)
        pltpu.sync_copy(dst_ref.at[r], dx)
        for k in range(_CHUNK // 16):
            sx[pl.ds(16 * k, 16)] = sx[pl.ds(16 * k, 16)] + noff
            dg[pl.ds(16 * k, 16)] = dx[pl.ds(16 * k, 16)] + noff
        pltpu.async_copy(tl_ref.at[sx], ab, sa)
        pltpu.async_copy(tr_ref.at[dg], bb, sb)

    def _wait(ab, bb, sa, sb):
        pltpu.make_async_copy(tl_ref.at[pl.ds(0, _CHUNK)], ab, sa).wait()
        pltpu.make_async_copy(tr_ref.at[pl.ds(0, _CHUNK)], bb, sb).wait()

    def _compute(ci, ab, bb, eb, dx):
        def _edge(j, _):
            for jj in (2 * j, 2 * j + 1):
                e = ab[jj, :] + bb[jj, :]
                e = jnp.maximum(e, _LEAK * e) - cvec
                eb[jj, :] = jnp.exp(e)
            return 0
        lax.fori_loop(0, _CHUNK // 2, _edge, 0)
        r = cid * _R + sid + _NS * ci
        pltpu.sync_copy(eb, ee_ref.at[r])
        pltpu.sync_copy(eb, dsh.at[dx], add=True)

    _issue(0, sx0, dx0, dg0, ab0, bb0, sa0, sb0)

    def _pair(i, _):
        c0 = 2 * i
        c1 = 2 * i + 1
        c2 = 2 * i + 2

        @pl.when(c1 < nb)
        def _i1():
            _issue(c1, sx1, dx1, dg1, ab1, bb1, sa1, sb1)

        @pl.when(c0 < nb)
        def _c0():
            _wait(ab0, bb0, sa0, sb0)
            _compute(c0, ab0, bb0, eb0, dx0)

        @pl.when(c2 < nb)
        def _i2():
            _issue(c2, sx0, dx0, dg0, ab0, bb0, sa0, sb0)

        @pl.when(c1 < nb)
        def _c1():
            _wait(ab1, bb1, sa1, sb1)
            _compute(c1, ab1, bb1, eb1, dx1)
        return 0
    lax.fori_loop(0, _NPAIR, _pair, 0)

    plsc.subcore_barrier()
    _export_shared(dsh, lambda o, sz: dp_ref.at[cid, pl.ds(o, sz)], sid)


def _p1(src2, dst2, tl, tr, c):
    mesh = plsc.VectorSubcoreMesh(core_axis_name="c", subcore_axis_name="s")
    f = pl.kernel(
        _p1_body,
        out_type=[
            jax.ShapeDtypeStruct((_NC * _R, _CHUNK, 16), jnp.float32),
            jax.ShapeDtypeStruct((_NC, _N, 16), jnp.float32),
        ],
        mesh=mesh,
        compiler_params=pltpu.CompilerParams(use_tc_tiling_on_sc=False),
        scratch_types=[
            pltpu.VMEM((_CHUNK,), jnp.int32),
            pltpu.VMEM((_CHUNK,), jnp.int32),
            pltpu.VMEM((_CHUNK,), jnp.int32),
            pltpu.VMEM((_CHUNK,), jnp.int32),
            pltpu.VMEM((_CHUNK,), jnp.int32),
            pltpu.VMEM((_CHUNK,), jnp.int32),
            pltpu.VMEM((_CHUNK, 16), jnp.float32),
            pltpu.VMEM((_CHUNK, 16), jnp.float32),
            pltpu.VMEM((_CHUNK, 16), jnp.float32),
            pltpu.VMEM((_CHUNK, 16), jnp.float32),
            pltpu.VMEM((_CHUNK, 16), jnp.float32),
            pltpu.VMEM((_CHUNK, 16), jnp.float32),
            pltpu.VMEM((_NC, 16), jnp.float32),
            pltpu.VMEM_SHARED((_N, 16), jnp.float32),
            pltpu.SemaphoreType.DMA,
            pltpu.SemaphoreType.DMA,
            pltpu.SemaphoreType.DMA,
            pltpu.SemaphoreType.DMA,
        ],
    )
    return f(src2, dst2, tl, tr, c)


def _p2_body(src_ref, dst_ref, feat_ref, ee_ref,
             op_ref,
             sx0, dx0, sx1, dx1, fb0, fb1, eb0, eb1, osh,
             sa0, sa1, so0, so1):
    cid = lax.axis_index("c")
    sid = lax.axis_index("s")

    def _z(i, _):
        for h in range(_H):
            fb0[i, pl.ds(16 * h, 16)] = jnp.zeros((16,), jnp.float32)
        return 0
    lax.fori_loop(0, _CHUNK, _z, 0)
    _zero_shared(fb0, osh, sid, _D)
    plsc.subcore_barrier()

    noff = cid * _N
    nb = _NB_BASE + jnp.where(sid < _NB_EXTRA, 1, 0)

    def _issue(ci, first, sx, dx, fb, eb, sa, so):
        @pl.when(jnp.logical_not(first))
        def _dr():
            pltpu.make_async_copy(fb, osh.at[pl.ds(0, _CHUNK)], so).wait()
        r = cid * _R + sid + _NS * ci
        pltpu.sync_copy(src_ref.at[r], sx)
        pltpu.sync_copy(dst_ref.at[r], dx)
        for k in range(_CHUNK // 16):
            sx[pl.ds(16 * k, 16)] = sx[pl.ds(16 * k, 16)] + noff
        pltpu.async_copy(feat_ref.at[sx], fb, sa)
        pltpu.sync_copy(ee_ref.at[r], eb)

    def _compute(fb, eb, dx, sa, so):
        pltpu.make_async_copy(feat_ref.at[pl.ds(0, _CHUNK)], fb, sa).wait()

        def _edge(j, _):
            for jj in (2 * j, 2 * j + 1):
                a = eb[jj, :]
                for h in range(_H):
                    ah = a.at[jnp.full((16,), h, jnp.int32)].get(
                        mode="promise_in_bounds")
                    fb[jj, pl.ds(16 * h, 16)] = (
                        fb[jj, pl.ds(16 * h, 16)] * ah)
            return 0
        lax.fori_loop(0, _CHUNK // 2, _edge, 0)
        pltpu.async_copy(fb, osh.at[dx], so, add=True)

    _issue(0, True, sx0, dx0, fb0, eb0, sa0, so0)

    def _pair(i, _):
        c0 = 2 * i
        c1 = 2 * i + 1
        c2 = 2 * i + 2

        @pl.when(c1 < nb)
        def _i1():
            _issue(c1, i == 0, sx1, dx1, fb1, eb1, sa1, so1)

        @pl.when(c0 < nb)
        def _c0():
            _compute(fb0, eb0, dx0, sa0, so0)

        @pl.when(c2 < nb)
        def _i2():
            _issue(c2, False, sx0, dx0, fb0, eb0, sa0, so0)

        @pl.when(c1 < nb)
        def _c1():
            _compute(fb1, eb1, dx1, sa1, so1)
        return 0
    lax.fori_loop(0, _NPAIR, _pair, 0)

    @pl.when(nb >= 1)
    def _dr0():
        pltpu.make_async_copy(fb0, osh.at[pl.ds(0, _CHUNK)], so0).wait()

    @pl.when(nb >= 2)
    def _dr1():
        pltpu.make_async_copy(fb1, osh.at[pl.ds(0, _CHUNK)], so1).wait()
    plsc.subcore_barrier()
    _export_shared(osh, lambda o, sz: op_ref.at[cid, pl.ds(o, sz)], sid)


def _p2(src2, dst2, feat, ee):
    mesh = plsc.VectorSubcoreMesh(core_axis_name="c", subcore_axis_name="s")
    f = pl.kernel(
        _p2_body,
        out_type=jax.ShapeDtypeStruct((_NC, _N, _D), jnp.float32),
        mesh=mesh,
        compiler_params=pltpu.CompilerParams(use_tc_tiling_on_sc=False),
        scratch_types=[
            pltpu.VMEM((_CHUNK,), jnp.int32),
            pltpu.VMEM((_CHUNK,), jnp.int32),
            pltpu.VMEM((_CHUNK,), jnp.int32),
            pltpu.VMEM((_CHUNK,), jnp.int32),
            pltpu.VMEM((_CHUNK, _D), jnp.float32),
            pltpu.VMEM((_CHUNK, _D), jnp.float32),
            pltpu.VMEM((_CHUNK, 16), jnp.float32),
            pltpu.VMEM((_CHUNK, 16), jnp.float32),
            pltpu.VMEM_SHARED((_N, _D), jnp.float32),
            pltpu.SemaphoreType.DMA,
            pltpu.SemaphoreType.DMA,
            pltpu.SemaphoreType.DMA,
            pltpu.SemaphoreType.DMA,
        ],
    )
    return f(src2, dst2, feat, ee)


def _ka_body(op_ref, dp_ref, bb_ref, wp1_ref, bp1_ref, wp2_ref,
             z0_ref, z1_ref, w_ref):
    lane = lax.broadcasted_iota(jnp.int32, (8, _D), 1)
    row = lax.broadcasted_iota(jnp.int32, (8, _D), 0)
    eh = jnp.where(lane // _OUT == row, 1.0, 0.0)

    r0 = 1.0 / (dp_ref[0][:, 0:8] + 1e-30)
    r1 = 1.0 / (dp_ref[1][:, 0:8] + 1e-30)
    s0 = jnp.dot(r0, eh, preferred_element_type=jnp.float32)
    s1 = jnp.dot(r1, eh, preferred_element_type=jnp.float32)
    z0 = op_ref[0] * s0 + bb_ref[0:1, :]
    z1 = op_ref[1] * s1 + bb_ref[1:2, :]
    z0_ref[...] = z0
    z1_ref[...] = z1
    t0 = jnp.tanh(jnp.dot(z0, wp1_ref[...], preferred_element_type=jnp.float32)
                  + bp1_ref[...])
    t1 = jnp.tanh(jnp.dot(z1, wp1_ref[...], preferred_element_type=jnp.float32)
                  + bp1_ref[...])
    w0 = jnp.sum(t0 * wp2_ref[...], axis=1, keepdims=True)
    w1 = jnp.sum(t1 * wp2_ref[...], axis=1, keepdims=True)
    w_ref[...] = jnp.concatenate([w0, w1], axis=1)


def _ka(op, dp, bb, wp1, bp1r, wp2r):
    blk = _BLK
    return pl.pallas_call(
        _ka_body,
        grid=(_GRID,),
        in_specs=[
            pl.BlockSpec((_NC, blk, _D), lambda i: (0, i, 0)),
            pl.BlockSpec((_NC, blk, 16), lambda i: (0, i, 0)),
            pl.BlockSpec((_NC, _D), lambda i: (0, 0)),
            pl.BlockSpec((_D, _HID), lambda i: (0, 0)),
            pl.BlockSpec((1, _HID), lambda i: (0, 0)),
            pl.BlockSpec((1, _HID), lambda i: (0, 0)),
        ],
        out_specs=[
            pl.BlockSpec((blk, _D), lambda i: (i, 0)),
            pl.BlockSpec((blk, _D), lambda i: (i, 0)),
            pl.BlockSpec((blk, 2), lambda i: (i, 0)),
        ],
        out_shape=[
            jax.ShapeDtypeStruct((_N, _D), jnp.float32),
            jax.ShapeDtypeStruct((_N, _D), jnp.float32),
            jax.ShapeDtypeStruct((_N, 2), jnp.float32),
        ],
    )(op, dp, bb, wp1, bp1r, wp2r)


def _kb_body(z0_ref, z1_ref, w_ref, out_ref):
    w = w_ref[...]
    s0 = jnp.sum(w[:, 0:1]) / _N
    s1 = jnp.sum(w[:, 1:2]) / _N
    m = jnp.maximum(s0, s1)
    e0 = jnp.exp(s0 - m)
    e1 = jnp.exp(s1 - m)
    beta0 = e0 / (e0 + e1)
    beta1 = e1 / (e0 + e1)
    out_ref[...] = beta0 * z0_ref[...] + beta1 * z1_ref[...]


def _kb(z0, z1, w):
    blk = _BLK
    return pl.pallas_call(
        _kb_body,
        grid=(_GRID,),
        in_specs=[
            pl.BlockSpec((blk, _D), lambda i: (i, 0)),
            pl.BlockSpec((blk, _D), lambda i: (i, 0)),
            pl.BlockSpec((_N, 2), lambda i: (0, 0)),
        ],
        out_specs=pl.BlockSpec((blk, _D), lambda i: (i, 0)),
        out_shape=jax.ShapeDtypeStruct((_N, _D), jnp.float32),
    )(z0, z1, w)


def _attn_mats(attn_l, attn_r):
    k = jnp.arange(_D)
    hp = jnp.arange(16)
    mask = (k[:, None] // _OUT) == (hp[None, :] % _H)
    ml = jnp.where(mask, attn_l.reshape(_D)[:, None], 0.0)
    mr = jnp.where(mask, attn_r.reshape(_D)[:, None], 0.0)
    return ml.astype(jnp.float32), mr.astype(jnp.float32)


def kernel(x, edge_index_0, edge_index_1, W0, attn_l0, attn_r0, b0,
           W1, attn_l1, attn_r1, b1, Wp1, bp1, Wp2):
    src2 = jnp.stack([edge_index_0[0], edge_index_1[0]]).reshape(_NC * _R, _CHUNK)
    dst2 = jnp.stack([edge_index_0[1], edge_index_1[1]]).reshape(_NC * _R, _CHUNK)

    ml0, mr0 = _attn_mats(attn_l0, attn_r0)
    ml1, mr1 = _attn_mats(attn_l1, attn_r1)
    w = jnp.stack([W0, W1])
    ml = jnp.stack([ml0, ml1])
    mr = jnp.stack([mr0, mr1])
    bb = jnp.stack([b0, b1])

    feat, tl, tr, c = _pre(x, w, ml, mr)

    ee, dp = _p1(src2, dst2, tl.reshape(_NC * _N, 16),
                 tr.reshape(_NC * _N, 16), c.reshape(_NC, 16))
    op = _p2(src2, dst2, feat.reshape(_NC * _N, _D), ee)

    z0, z1, wsem = _ka(op, dp, bb, Wp1, bp1.reshape(1, _HID),
                       Wp2.reshape(1, _HID))
    return _kb(z0, z1, wsem)


# speedup vs baseline: 1.1746x; 1.1746x over previous
import jax
import jax.numpy as jnp
from jax import lax
from jax.experimental import pallas as pl
from jax.experimental.pallas import tpu as pltpu
from jax.experimental.pallas import tpu_sc as plsc

_N = 10000
_E = 320000
_IN = 128
_H = 8
_OUT = 16
_D = _H * _OUT
_HID = 128
_CHUNK = 128
_R = _E // _CHUNK
_NC = 2
_NS = 16
_SUB_BASE = 624
_SUB_CHUNKS = ((0, 128), (128, 128), (256, 128), (384, 128), (512, 112))
_TAIL_OFF = _SUB_BASE * _NS
_TAIL = _N - _TAIL_OFF
_LEAK = 0.2
_BLK = 2000
_GRID = _N // _BLK
_NB_BASE = _R // _NS
_NB_EXTRA = _R - _NB_BASE * _NS
_NPAIR = (_NB_BASE + _NB_EXTRA + 1) // 2


def _pre_body(x_ref, w_ref, ml_ref, mr_ref,
              feat_ref, tl_ref, tr_ref, c_ref, acc_ref):
    i = pl.program_id(1)
    x = x_ref[...]
    f = jnp.dot(x, w_ref[0], preferred_element_type=jnp.float32)
    feat_ref[0] = f
    tl = jnp.dot(f, ml_ref[0], preferred_element_type=jnp.float32)
    tr = jnp.dot(f, mr_ref[0], preferred_element_type=jnp.float32)
    tl_ref[0] = tl
    tr_ref[0] = tr
    for row, t in enumerate((tl, tr)):
        m = jnp.max(t, axis=0)
        prev = acc_ref[row, :]
        acc_ref[row, :] = jnp.where(i == 0, m, jnp.maximum(prev, m))
    zero = jnp.zeros((16,), jnp.float32)
    c_ref[0, 0, :] = jnp.maximum(zero, acc_ref[0, :] + acc_ref[1, :])


def _pre(x, w, ml, mr):
    blk = _BLK
    return pl.pallas_call(
        _pre_body,
        grid=(_NC, _GRID),
        in_specs=[
            pl.BlockSpec((blk, _IN), lambda p, i: (i, 0)),
            pl.BlockSpec((1, _IN, _D), lambda p, i: (p, 0, 0)),
            pl.BlockSpec((1, _D, 16), lambda p, i: (p, 0, 0)),
            pl.BlockSpec((1, _D, 16), lambda p, i: (p, 0, 0)),
        ],
        out_specs=[
            pl.BlockSpec((1, blk, _D), lambda p, i: (p, i, 0)),
            pl.BlockSpec((1, blk, 16), lambda p, i: (p, i, 0)),
            pl.BlockSpec((1, blk, 16), lambda p, i: (p, i, 0)),
            pl.BlockSpec((1, 1, 16), lambda p, i: (p, 0, 0)),
        ],
        out_shape=[
            jax.ShapeDtypeStruct((_NC, _N, _D), jnp.float32),
            jax.ShapeDtypeStruct((_NC, _N, 16), jnp.float32),
            jax.ShapeDtypeStruct((_NC, _N, 16), jnp.float32),
            jax.ShapeDtypeStruct((_NC, 1, 16), jnp.float32),
        ],
        scratch_shapes=[pltpu.VMEM((2, 16), jnp.float32)],
    )(x, w, ml, mr)


def _zero_shared(zbuf, sh, sid, width):
    base = pl.multiple_of(sid * _SUB_BASE, 8)
    for off, sz in _SUB_CHUNKS:
        pltpu.sync_copy(zbuf.at[pl.ds(0, sz)],
                        sh.at[pl.ds(pl.multiple_of(base + off, 8), sz)])

    @pl.when(sid == _NS - 1)
    def _zt():
        pltpu.sync_copy(zbuf.at[pl.ds(0, _TAIL)], sh.at[pl.ds(_TAIL_OFF, _TAIL)])


def _export_shared(sh, out2d_at_cid, sid):
    base = pl.multiple_of(sid * _SUB_BASE, 8)
    for off, sz in _SUB_CHUNKS:
        o = pl.multiple_of(base + off, 8)
        pltpu.sync_copy(sh.at[pl.ds(o, sz)], out2d_at_cid(o, sz))

    @pl.when(sid == _NS - 1)
    def _xt():
        pltpu.sync_copy(sh.at[pl.ds(_TAIL_OFF, _TAIL)],
                        out2d_at_cid(_TAIL_OFF, _TAIL))


def _p1_body(src_ref, dst_ref, tl_ref, tr_ref, c_ref,
             ee_ref, dp_ref,
             sx0, dx0, dg0, sx1, dx1, dg1,
             ab0, bb0, ab1, bb1, eb0, eb1,
             cbuf, dsh, sa0, sb0, sa1, sb1):
    cid = lax.axis_index("c")
    sid = lax.axis_index("s")

    def _z(i, _):
        eb0[i, :] = jnp.zeros((16,), jnp.float32)
        return 0
    lax.fori_loop(0, _CHUNK, _z, 0)
    _zero_shared(eb0, dsh, sid, 16)
    plsc.subcore_barrier()

    pltpu.sync_copy(c_ref, cbuf)
    cvec = cbuf[cid, :]
    noff = cid * _N

    nb = _NB_BASE + jnp.where(sid < _NB_EXTRA, 1, 0)

    def _issue(ci, sx, dx, dg, ab, bb, sa, sb):
        r = cid * _R + sid + _NS * ci
        pltpu.sync_copy(src_ref.at[r], sx)
        pltpu.sync_copy(dst_ref.at[r], dx)
        for k in range(_CHUNK // 16):
            sx[pl.ds(16 * k, 16)] = sx[pl.ds(16 * k, 16)] + noff
            dg[pl.ds(16 * k, 16)] = dx[pl.ds(16 * k, 16)] + noff
        pltpu.async_copy(tl_ref.at[sx], ab, sa)
        pltpu.async_copy(tr_ref.at[dg], bb, sb)

    def _wait(ab, bb, sa, sb):
        pltpu.make_async_copy(tl_ref.at[pl.ds(0, _CHUNK)], ab, sa).wait()
        pltpu.make_async_copy(tr_ref.at[pl.ds(0, _CHUNK)], bb, sb).wait()

    def _compute(ci, ab, bb, eb, dx):
        def _edge(j, _):
            for jj in (2 * j, 2 * j + 1):
                e = ab[jj, :] + bb[jj, :]
                e = jnp.maximum(e, _LEAK * e) - cvec
                eb[jj, :] = jnp.exp(e)
            return 0
        lax.fori_loop(0, _CHUNK // 2, _edge, 0)
        r = cid * _R + sid + _NS * ci
        pltpu.sync_copy(eb, ee_ref.at[r])
        pltpu.sync_copy(eb, dsh.at[dx], add=True)

    _issue(0, sx0, dx0, dg0, ab0, bb0, sa0, sb0)

    def _pair(i, _):
        c0 = 2 * i
        c1 = 2 * i + 1
        c2 = 2 * i + 2

        @pl.when(c1 < nb)
        def _i1():
            _issue(c1, sx1, dx1, dg1, ab1, bb1, sa1, sb1)

        @pl.when(c0 < nb)
        def _c0():
            _wait(ab0, bb0, sa0, sb0)
            _compute(c0, ab0, bb0, eb0, dx0)

        @pl.when(c2 < nb)
        def _i2():
            _issue(c2, sx0, dx0, dg0, ab0, bb0, sa0, sb0)

        @pl.when(c1 < nb)
        def _c1():
            _wait(ab1, bb1, sa1, sb1)
            _compute(c1, ab1, bb1, eb1, dx1)
        return 0
    lax.fori_loop(0, _NPAIR, _pair, 0)

    plsc.subcore_barrier()
    _export_shared(dsh, lambda o, sz: dp_ref.at[cid, pl.ds(o, sz)], sid)


def _p1(src2, dst2, tl, tr, c):
    mesh = plsc.VectorSubcoreMesh(core_axis_name="c", subcore_axis_name="s")
    f = pl.kernel(
        _p1_body,
        out_type=[
            jax.ShapeDtypeStruct((_NC * _R, _CHUNK, 16), jnp.float32),
            jax.ShapeDtypeStruct((_NC, _N, 16), jnp.float32),
        ],
        mesh=mesh,
        compiler_params=pltpu.CompilerParams(use_tc_tiling_on_sc=False),
        scratch_types=[
            pltpu.VMEM((_CHUNK,), jnp.int32),
            pltpu.VMEM((_CHUNK,), jnp.int32),
            pltpu.VMEM((_CHUNK,), jnp.int32),
            pltpu.VMEM((_CHUNK,), jnp.int32),
            pltpu.VMEM((_CHUNK,), jnp.int32),
            pltpu.VMEM((_CHUNK,), jnp.int32),
            pltpu.VMEM((_CHUNK, 16), jnp.float32),
            pltpu.VMEM((_CHUNK, 16), jnp.float32),
            pltpu.VMEM((_CHUNK, 16), jnp.float32),
            pltpu.VMEM((_CHUNK, 16), jnp.float32),
            pltpu.VMEM((_CHUNK, 16), jnp.float32),
            pltpu.VMEM((_CHUNK, 16), jnp.float32),
            pltpu.VMEM((_NC, 16), jnp.float32),
            pltpu.VMEM_SHARED((_N, 16), jnp.float32),
            pltpu.SemaphoreType.DMA,
            pltpu.SemaphoreType.DMA,
            pltpu.SemaphoreType.DMA,
            pltpu.SemaphoreType.DMA,
        ],
    )
    return f(src2, dst2, tl, tr, c)


def _p2_body(src_ref, dst_ref, feat_ref, ee_ref,
             op_ref,
             sxA, sxB, sxC, dxA, dxB, dxC, ebA, ebB, ebC, fb0, fb1, osh,
             siA, siB, siC, seA, seB, seC, sf0, sf1, so0, so1):
    cid = lax.axis_index("c")
    sid = lax.axis_index("s")
    sxs = (sxA, sxB, sxC)
    dxs = (dxA, dxB, dxC)
    ebs = (ebA, ebB, ebC)
    sis = (siA, siB, siC)
    ses = (seA, seB, seC)
    fbs = (fb0, fb1)
    sfs = (sf0, sf1)
    sos = (so0, so1)

    def _z(i, _):
        for h in range(_H):
            fb0[i, pl.ds(16 * h, 16)] = jnp.zeros((16,), jnp.float32)
        return 0
    lax.fori_loop(0, _CHUNK, _z, 0)
    _zero_shared(fb0, osh, sid, _D)
    plsc.subcore_barrier()

    noff = cid * _N
    nb = _NB_BASE + jnp.where(sid < _NB_EXTRA, 1, 0)

    def _pf(ci, k):
        r = cid * _R + sid + _NS * ci
        pltpu.async_copy(src_ref.at[r], sxs[k], sis[k])
        pltpu.async_copy(dst_ref.at[r], dxs[k], sis[k])
        pltpu.async_copy(ee_ref.at[r], ebs[k], ses[k])

    def _launch(ci, k, kf):
        @pl.when(ci >= 2)
        def _dr():
            pltpu.make_async_copy(fbs[kf], osh.at[pl.ds(0, _CHUNK)],
                                  sos[kf]).wait()
        pltpu.make_async_copy(src_ref.at[0], sxs[k], sis[k]).wait()
        pltpu.make_async_copy(src_ref.at[0], sxs[k], sis[k]).wait()
        sx = sxs[k]
        for kk in range(_CHUNK // 16):
            sx[pl.ds(16 * kk, 16)] = sx[pl.ds(16 * kk, 16)] + noff
        pltpu.async_copy(feat_ref.at[sx], fbs[kf], sfs[kf])

    def _comp(k, kf):
        fb = fbs[kf]
        eb = ebs[k]
        pltpu.make_async_copy(feat_ref.at[pl.ds(0, _CHUNK)], fb, sfs[kf]).wait()
        pltpu.make_async_copy(ee_ref.at[0], eb, ses[k]).wait()

        def _edge(j, _):
            for jj in (2 * j, 2 * j + 1):
                a = eb[jj, :]
                for h in range(_H):
                    fb[jj, pl.ds(16 * h, 16)] = (
                        fb[jj, pl.ds(16 * h, 16)] * a[h])
            return 0
        lax.fori_loop(0, _CHUNK // 2, _edge, 0)
        pltpu.async_copy(fb, osh.at[dxs[k]], sos[kf], add=True)

    _pf(0, 0)
    _pf(1, 1)
    _launch(0, 0, 0)

    nsteps = (_NB_BASE + _NB_EXTRA + 5) // 6 + 1

    def _macro(i, _):
        for k in range(6):
            c = 6 * i + k

            @pl.when(c < nb)
            def _c():
                _comp(k % 3, k % 2)

            @pl.when(c + 1 < nb)
            def _l():
                _launch(c + 1, (k + 1) % 3, (k + 1) % 2)

            @pl.when(c + 2 < nb)
            def _p():
                _pf(c + 2, (k + 2) % 3)
        return 0
    lax.fori_loop(0, nsteps, _macro, 0)

    @pl.when(nb >= 1)
    def _dr0():
        pltpu.make_async_copy(fb0, osh.at[pl.ds(0, _CHUNK)], so0).wait()

    @pl.when(nb >= 2)
    def _dr1():
        pltpu.make_async_copy(fb1, osh.at[pl.ds(0, _CHUNK)], so1).wait()
    plsc.subcore_barrier()
    _export_shared(osh, lambda o, sz: op_ref.at[cid, pl.ds(o, sz)], sid)


def _p2(src2, dst2, feat, ee):
    mesh = plsc.VectorSubcoreMesh(core_axis_name="c", subcore_axis_name="s")
    f = pl.kernel(
        _p2_body,
        out_type=jax.ShapeDtypeStruct((_NC, _N, _D), jnp.float32),
        mesh=mesh,
        compiler_params=pltpu.CompilerParams(use_tc_tiling_on_sc=False),
        scratch_types=(
            [pltpu.VMEM((_CHUNK,), jnp.int32)] * 6
            + [pltpu.VMEM((_CHUNK, 16), jnp.float32)] * 3
            + [pltpu.VMEM((_CHUNK, _D), jnp.float32)] * 2
            + [pltpu.VMEM_SHARED((_N, _D), jnp.float32)]
            + [pltpu.SemaphoreType.DMA] * 10
        ),
    )
    return f(src2, dst2, feat, ee)


def _ka_body(op_ref, dp_ref, bb_ref, wp1_ref, bp1_ref, wp2_ref,
             z0_ref, z1_ref, w_ref):
    lane = lax.broadcasted_iota(jnp.int32, (8, _D), 1)
    row = lax.broadcasted_iota(jnp.int32, (8, _D), 0)
    eh = jnp.where(lane // _OUT == row, 1.0, 0.0)

    r0 = 1.0 / (dp_ref[0][:, 0:8] + 1e-30)
    r1 = 1.0 / (dp_ref[1][:, 0:8] + 1e-30)
    s0 = jnp.dot(r0, eh, preferred_element_type=jnp.float32)
    s1 = jnp.dot(r1, eh, preferred_element_type=jnp.float32)
    z0 = op_ref[0] * s0 + bb_ref[0:1, :]
    z1 = op_ref[1] * s1 + bb_ref[1:2, :]
    z0_ref[...] = z0
    z1_ref[...] = z1
    t0 = jnp.tanh(jnp.dot(z0, wp1_ref[...], preferred_element_type=jnp.float32)
                  + bp1_ref[...])
    t1 = jnp.tanh(jnp.dot(z1, wp1_ref[...], preferred_element_type=jnp.float32)
                  + bp1_ref[...])
    w0 = jnp.sum(t0 * wp2_ref[...], axis=1, keepdims=True)
    w1 = jnp.sum(t1 * wp2_ref[...], axis=1, keepdims=True)
    w_ref[...] = jnp.concatenate([w0, w1], axis=1)


def _ka(op, dp, bb, wp1, bp1r, wp2r):
    blk = _BLK
    return pl.pallas_call(
        _ka_body,
        grid=(_GRID,),
        in_specs=[
            pl.BlockSpec((_NC, blk, _D), lambda i: (0, i, 0)),
            pl.BlockSpec((_NC, blk, 16), lambda i: (0, i, 0)),
            pl.BlockSpec((_NC, _D), lambda i: (0, 0)),
            pl.BlockSpec((_D, _HID), lambda i: (0, 0)),
            pl.BlockSpec((1, _HID), lambda i: (0, 0)),
            pl.BlockSpec((1, _HID), lambda i: (0, 0)),
        ],
        out_specs=[
            pl.BlockSpec((blk, _D), lambda i: (i, 0)),
            pl.BlockSpec((blk, _D), lambda i: (i, 0)),
            pl.BlockSpec((blk, 2), lambda i: (i, 0)),
        ],
        out_shape=[
            jax.ShapeDtypeStruct((_N, _D), jnp.float32),
            jax.ShapeDtypeStruct((_N, _D), jnp.float32),
            jax.ShapeDtypeStruct((_N, 2), jnp.float32),
        ],
    )(op, dp, bb, wp1, bp1r, wp2r)


def _kb_body(z0_ref, z1_ref, w_ref, out_ref):
    w = w_ref[...]
    s0 = jnp.sum(w[:, 0:1]) / _N
    s1 = jnp.sum(w[:, 1:2]) / _N
    m = jnp.maximum(s0, s1)
    e0 = jnp.exp(s0 - m)
    e1 = jnp.exp(s1 - m)
    beta0 = e0 / (e0 + e1)
    beta1 = e1 / (e0 + e1)
    out_ref[...] = beta0 * z0_ref[...] + beta1 * z1_ref[...]


def _kb(z0, z1, w):
    blk = _BLK
    return pl.pallas_call(
        _kb_body,
        grid=(_GRID,),
        in_specs=[
            pl.BlockSpec((blk, _D), lambda i: (i, 0)),
            pl.BlockSpec((blk, _D), lambda i: (i, 0)),
            pl.BlockSpec((_N, 2), lambda i: (0, 0)),
        ],
        out_specs=pl.BlockSpec((blk, _D), lambda i: (i, 0)),
        out_shape=jax.ShapeDtypeStruct((_N, _D), jnp.float32),
    )(z0, z1, w)


def _attn_mats(attn_l, attn_r):
    k = jnp.arange(_D)
    hp = jnp.arange(16)
    mask = (k[:, None] // _OUT) == (hp[None, :] % _H)
    ml = jnp.where(mask, attn_l.reshape(_D)[:, None], 0.0)
    mr = jnp.where(mask, attn_r.reshape(_D)[:, None], 0.0)
    return ml.astype(jnp.float32), mr.astype(jnp.float32)


def kernel(x, edge_index_0, edge_index_1, W0, attn_l0, attn_r0, b0,
           W1, attn_l1, attn_r1, b1, Wp1, bp1, Wp2):
    src2 = jnp.stack([edge_index_0[0], edge_index_1[0]]).reshape(_NC * _R, _CHUNK)
    dst2 = jnp.stack([edge_index_0[1], edge_index_1[1]]).reshape(_NC * _R, _CHUNK)

    ml0, mr0 = _attn_mats(attn_l0, attn_r0)
    ml1, mr1 = _attn_mats(attn_l1, attn_r1)
    w = jnp.stack([W0, W1])
    ml = jnp.stack([ml0, ml1])
    mr = jnp.stack([mr0, mr1])
    bb = jnp.stack([b0, b1])

    feat, tl, tr, c = _pre(x, w, ml, mr)

    ee, dp = _p1(src2, dst2, tl.reshape(_NC * _N, 16),
                 tr.reshape(_NC * _N, 16), c.reshape(_NC, 16))
    op = _p2(src2, dst2, feat.reshape(_NC * _N, _D), ee)

    z0, z1, wsem = _ka(op, dp, bb, Wp1, bp1.reshape(1, _HID),
                       Wp2.reshape(1, _HID))
    return _kb(z0, z1, wsem)


# speedup vs baseline: 1.2605x; 1.0731x over previous
import jax
import jax.numpy as jnp
from jax import lax
from jax.experimental import pallas as pl
from jax.experimental.pallas import tpu as pltpu
from jax.experimental.pallas import tpu_sc as plsc

_N = 10000
_E = 320000
_IN = 128
_H = 8
_OUT = 16
_D = _H * _OUT
_HID = 128
_CHUNK = 128
_R = _E // _CHUNK
_NC = 2
_NS = 16
_SUB_BASE = 624
_SUB_CHUNKS = ((0, 128), (128, 128), (256, 128), (384, 128), (512, 112))
_TAIL_OFF = _SUB_BASE * _NS
_TAIL = _N - _TAIL_OFF
_LEAK = 0.2
_BLK = 2000
_GRID = _N // _BLK
_NB_BASE = _R // _NS
_NB_EXTRA = _R - _NB_BASE * _NS
_NPAIR = (_NB_BASE + _NB_EXTRA + 1) // 2


def _pre_body(x_ref, w_ref, ml_ref, mr_ref,
              feat_ref, tl_ref, tr_ref, c_ref, acc_ref):
    i = pl.program_id(1)
    x = x_ref[...]
    f = jnp.dot(x, w_ref[0], preferred_element_type=jnp.float32)
    feat_ref[0] = f
    tl = jnp.dot(f, ml_ref[0], preferred_element_type=jnp.float32)
    tr = jnp.dot(f, mr_ref[0], preferred_element_type=jnp.float32)
    tl_ref[0] = tl
    tr_ref[0] = tr
    for row, t in enumerate((tl, tr)):
        m = jnp.max(t, axis=0)
        prev = acc_ref[row, :]
        acc_ref[row, :] = jnp.where(i == 0, m, jnp.maximum(prev, m))
    zero = jnp.zeros((16,), jnp.float32)
    c_ref[0, 0, :] = jnp.maximum(zero, acc_ref[0, :] + acc_ref[1, :])


def _pre(x, w, ml, mr):
    blk = _BLK
    return pl.pallas_call(
        _pre_body,
        grid=(_NC, _GRID),
        in_specs=[
            pl.BlockSpec((blk, _IN), lambda p, i: (i, 0)),
            pl.BlockSpec((1, _IN, _D), lambda p, i: (p, 0, 0)),
            pl.BlockSpec((1, _D, 16), lambda p, i: (p, 0, 0)),
            pl.BlockSpec((1, _D, 16), lambda p, i: (p, 0, 0)),
        ],
        out_specs=[
            pl.BlockSpec((1, blk, _D), lambda p, i: (p, i, 0)),
            pl.BlockSpec((1, blk, 16), lambda p, i: (p, i, 0)),
            pl.BlockSpec((1, blk, 16), lambda p, i: (p, i, 0)),
            pl.BlockSpec((1, 1, 16), lambda p, i: (p, 0, 0)),
        ],
        out_shape=[
            jax.ShapeDtypeStruct((_NC, _N, _D), jnp.float32),
            jax.ShapeDtypeStruct((_NC, _N, 16), jnp.float32),
            jax.ShapeDtypeStruct((_NC, _N, 16), jnp.float32),
            jax.ShapeDtypeStruct((_NC, 1, 16), jnp.float32),
        ],
        scratch_shapes=[pltpu.VMEM((2, 16), jnp.float32)],
    )(x, w, ml, mr)


def _zero_shared(zbuf, sh, sid, width):
    base = pl.multiple_of(sid * _SUB_BASE, 8)
    for off, sz in _SUB_CHUNKS:
        pltpu.sync_copy(zbuf.at[pl.ds(0, sz)],
                        sh.at[pl.ds(pl.multiple_of(base + off, 8), sz)])

    @pl.when(sid == _NS - 1)
    def _zt():
        pltpu.sync_copy(zbuf.at[pl.ds(0, _TAIL)], sh.at[pl.ds(_TAIL_OFF, _TAIL)])


def _export_shared(sh, out2d_at_cid, sid):
    base = pl.multiple_of(sid * _SUB_BASE, 8)
    for off, sz in _SUB_CHUNKS:
        o = pl.multiple_of(base + off, 8)
        pltpu.sync_copy(sh.at[pl.ds(o, sz)], out2d_at_cid(o, sz))

    @pl.when(sid == _NS - 1)
    def _xt():
        pltpu.sync_copy(sh.at[pl.ds(_TAIL_OFF, _TAIL)],
                        out2d_at_cid(_TAIL_OFF, _TAIL))


def _p1_body(src_ref, dst_ref, tl_ref, tr_ref, c_ref,
             ee_ref, dp_ref,
             sxA, sxB, sxC, sxD, dxA, dxB, dxC, dxD, dgA, dgB, dgC, dgD,
             ab0, bb0, ab1, bb1, eb0, eb1, cbuf, dsh,
             siA, siB, siC, siD, sa0, sa1, sb0, sb1, eo0, eo1, sd0, sd1):
    cid = lax.axis_index("c")
    sid = lax.axis_index("s")
    sxs = (sxA, sxB, sxC, sxD)
    dxs = (dxA, dxB, dxC, dxD)
    dgs = (dgA, dgB, dgC, dgD)
    sis = (siA, siB, siC, siD)
    abs_ = (ab0, ab1)
    bbs = (bb0, bb1)
    ebs = (eb0, eb1)
    sas = (sa0, sa1)
    sbs = (sb0, sb1)
    eos = (eo0, eo1)
    sds = (sd0, sd1)

    def _z(i, _):
        eb0[i, :] = jnp.zeros((16,), jnp.float32)
        return 0
    lax.fori_loop(0, _CHUNK, _z, 0)
    _zero_shared(eb0, dsh, sid, 16)
    plsc.subcore_barrier()

    pltpu.sync_copy(c_ref, cbuf)
    cvec = cbuf[cid, :]
    noff = cid * _N

    nb = _NB_BASE + jnp.where(sid < _NB_EXTRA, 1, 0)

    def _pf(ci, k):
        r = cid * _R + sid + _NS * ci
        pltpu.async_copy(src_ref.at[r], sxs[k], sis[k])
        pltpu.async_copy(dst_ref.at[r], dxs[k], sis[k])

    def _launch(k, kf):
        pltpu.make_async_copy(src_ref.at[0], sxs[k], sis[k]).wait()
        pltpu.make_async_copy(src_ref.at[0], sxs[k], sis[k]).wait()
        sx = sxs[k]
        dx = dxs[k]
        dg = dgs[k]
        for kk in range(_CHUNK // 16):
            sx[pl.ds(16 * kk, 16)] = sx[pl.ds(16 * kk, 16)] + noff
            dg[pl.ds(16 * kk, 16)] = dx[pl.ds(16 * kk, 16)] + noff
        pltpu.async_copy(tl_ref.at[sx], abs_[kf], sas[kf])
        pltpu.async_copy(tr_ref.at[dg], bbs[kf], sbs[kf])

    def _comp(ci, k, kf):
        ab = abs_[kf]
        bb = bbs[kf]
        eb = ebs[kf]
        pltpu.make_async_copy(tl_ref.at[pl.ds(0, _CHUNK)], ab, sas[kf]).wait()
        pltpu.make_async_copy(tr_ref.at[pl.ds(0, _CHUNK)], bb, sbs[kf]).wait()

        @pl.when(ci >= 2)
        def _dr():
            pltpu.make_async_copy(eb, ee_ref.at[0], eos[kf]).wait()
            pltpu.make_async_copy(ab, dsh.at[pl.ds(0, _CHUNK)], sds[kf]).wait()

        def _edge(j, _):
            for jj in (2 * j, 2 * j + 1):
                e = ab[jj, :] + bb[jj, :]
                e = jnp.maximum(e, _LEAK * e) - cvec
                eb[jj, :] = jnp.exp(e)
            return 0
        lax.fori_loop(0, _CHUNK // 2, _edge, 0)
        r = cid * _R + sid + _NS * ci
        pltpu.async_copy(eb, ee_ref.at[r], eos[kf])
        pltpu.async_copy(eb, dsh.at[dxs[k]], sds[kf], add=True)

    _pf(0, 0)
    _pf(1, 1)
    _launch(0, 0)

    nsteps = (_NB_BASE + _NB_EXTRA + 3) // 4 + 1

    def _macro(i, _):
        for k in range(4):
            c = 4 * i + k

            @pl.when(c < nb)
            def _c():
                _comp(c, k % 4, k % 2)

            @pl.when(c + 1 < nb)
            def _l():
                _launch((k + 1) % 4, (k + 1) % 2)

            @pl.when(c + 2 < nb)
            def _p():
                _pf(c + 2, (k + 2) % 4)
        return 0
    lax.fori_loop(0, nsteps, _macro, 0)

    @pl.when(nb >= 1)
    def _dr0():
        pltpu.make_async_copy(eb0, ee_ref.at[0], eo0).wait()
        pltpu.make_async_copy(ab0, dsh.at[pl.ds(0, _CHUNK)], sd0).wait()

    @pl.when(nb >= 2)
    def _dr1():
        pltpu.make_async_copy(eb1, ee_ref.at[0], eo1).wait()
        pltpu.make_async_copy(ab1, dsh.at[pl.ds(0, _CHUNK)], sd1).wait()
    plsc.subcore_barrier()
    _export_shared(dsh, lambda o, sz: dp_ref.at[cid, pl.ds(o, sz)], sid)


def _p1(src2, dst2, tl, tr, c):
    mesh = plsc.VectorSubcoreMesh(core_axis_name="c", subcore_axis_name="s")
    f = pl.kernel(
        _p1_body,
        out_type=[
            jax.ShapeDtypeStruct((_NC * _R, _CHUNK, 16), jnp.float32),
            jax.ShapeDtypeStruct((_NC, _N, 16), jnp.float32),
        ],
        mesh=mesh,
        compiler_params=pltpu.CompilerParams(use_tc_tiling_on_sc=False),
        scratch_types=(
            [pltpu.VMEM((_CHUNK,), jnp.int32)] * 12
            + [pltpu.VMEM((_CHUNK, 16), jnp.float32)] * 6
            + [pltpu.VMEM((_NC, 16), jnp.float32)]
            + [pltpu.VMEM_SHARED((_N, 16), jnp.float32)]
            + [pltpu.SemaphoreType.DMA] * 12
        ),
    )
    return f(src2, dst2, tl, tr, c)


def _p2_body(src_ref, dst_ref, feat_ref, ee_ref,
             op_ref,
             sxA, sxB, sxC, dxA, dxB, dxC, ebA, ebB, ebC, fb0, fb1, osh,
             siA, siB, siC, seA, seB, seC, sf0, sf1, so0, so1):
    cid = lax.axis_index("c")
    sid = lax.axis_index("s")
    sxs = (sxA, sxB, sxC)
    dxs = (dxA, dxB, dxC)
    ebs = (ebA, ebB, ebC)
    sis = (siA, siB, siC)
    ses = (seA, seB, seC)
    fbs = (fb0, fb1)
    sfs = (sf0, sf1)
    sos = (so0, so1)

    def _z(i, _):
        for h in range(_H):
            fb0[i, pl.ds(16 * h, 16)] = jnp.zeros((16,), jnp.float32)
        return 0
    lax.fori_loop(0, _CHUNK, _z, 0)
    _zero_shared(fb0, osh, sid, _D)
    plsc.subcore_barrier()

    noff = cid * _N
    nb = _NB_BASE + jnp.where(sid < _NB_EXTRA, 1, 0)

    def _pf(ci, k):
        r = cid * _R + sid + _NS * ci
        pltpu.async_copy(src_ref.at[r], sxs[k], sis[k])
        pltpu.async_copy(dst_ref.at[r], dxs[k], sis[k])
        pltpu.async_copy(ee_ref.at[r], ebs[k], ses[k])

    def _launch(ci, k, kf):
        @pl.when(ci >= 2)
        def _dr():
            pltpu.make_async_copy(fbs[kf], osh.at[pl.ds(0, _CHUNK)],
                                  sos[kf]).wait()
        pltpu.make_async_copy(src_ref.at[0], sxs[k], sis[k]).wait()
        pltpu.make_async_copy(src_ref.at[0], sxs[k], sis[k]).wait()
        sx = sxs[k]
        for kk in range(_CHUNK // 16):
            sx[pl.ds(16 * kk, 16)] = sx[pl.ds(16 * kk, 16)] + noff
        pltpu.async_copy(feat_ref.at[sx], fbs[kf], sfs[kf])

    def _comp(k, kf):
        fb = fbs[kf]
        eb = ebs[k]
        pltpu.make_async_copy(feat_ref.at[pl.ds(0, _CHUNK)], fb, sfs[kf]).wait()
        pltpu.make_async_copy(ee_ref.at[0], eb, ses[k]).wait()

        def _edge(j, _):
            for jj in (2 * j, 2 * j + 1):
                a = eb[jj, :]
                for h in range(_H):
                    fb[jj, pl.ds(16 * h, 16)] = (
                        fb[jj, pl.ds(16 * h, 16)] * a[h])
            return 0
        lax.fori_loop(0, _CHUNK // 2, _edge, 0)
        pltpu.async_copy(fb, osh.at[dxs[k]], sos[kf], add=True)

    _pf(0, 0)
    _pf(1, 1)
    _launch(0, 0, 0)

    nsteps = (_NB_BASE + _NB_EXTRA + 5) // 6 + 1

    def _macro(i, _):
        for k in range(6):
            c = 6 * i + k

            @pl.when(c < nb)
            def _c():
                _comp(k % 3, k % 2)

            @pl.when(c + 1 < nb)
            def _l():
                _launch(c + 1, (k + 1) % 3, (k + 1) % 2)

            @pl.when(c + 2 < nb)
            def _p():
                _pf(c + 2, (k + 2) % 3)
        return 0
    lax.fori_loop(0, nsteps, _macro, 0)

    @pl.when(nb >= 1)
    def _dr0():
        pltpu.make_async_copy(fb0, osh.at[pl.ds(0, _CHUNK)], so0).wait()

    @pl.when(nb >= 2)
    def _dr1():
        pltpu.make_async_copy(fb1, osh.at[pl.ds(0, _CHUNK)], so1).wait()
    plsc.subcore_barrier()
    _export_shared(osh, lambda o, sz: op_ref.at[cid, pl.ds(o, sz)], sid)


def _p2(src2, dst2, feat, ee):
    mesh = plsc.VectorSubcoreMesh(core_axis_name="c", subcore_axis_name="s")
    f = pl.kernel(
        _p2_body,
        out_type=jax.ShapeDtypeStruct((_NC, _N, _D), jnp.float32),
        mesh=mesh,
        compiler_params=pltpu.CompilerParams(use_tc_tiling_on_sc=False),
        scratch_types=(
            [pltpu.VMEM((_CHUNK,), jnp.int32)] * 6
            + [pltpu.VMEM((_CHUNK, 16), jnp.float32)] * 3
            + [pltpu.VMEM((_CHUNK, _D), jnp.float32)] * 2
            + [pltpu.VMEM_SHARED((_N, _D), jnp.float32)]
            + [pltpu.SemaphoreType.DMA] * 10
        ),
    )
    return f(src2, dst2, feat, ee)


def _ka_body(op_ref, dp_ref, bb_ref, wp1_ref, bp1_ref, wp2_ref,
             z0_ref, z1_ref, w_ref):
    lane = lax.broadcasted_iota(jnp.int32, (8, _D), 1)
    row = lax.broadcasted_iota(jnp.int32, (8, _D), 0)
    eh = jnp.where(lane // _OUT == row, 1.0, 0.0)

    r0 = 1.0 / (dp_ref[0][:, 0:8] + 1e-30)
    r1 = 1.0 / (dp_ref[1][:, 0:8] + 1e-30)
    s0 = jnp.dot(r0, eh, preferred_element_type=jnp.float32)
    s1 = jnp.dot(r1, eh, preferred_element_type=jnp.float32)
    z0 = op_ref[0] * s0 + bb_ref[0:1, :]
    z1 = op_ref[1] * s1 + bb_ref[1:2, :]
    z0_ref[...] = z0
    z1_ref[...] = z1
    t0 = jnp.tanh(jnp.dot(z0, wp1_ref[...], preferred_element_type=jnp.float32)
                  + bp1_ref[...])
    t1 = jnp.tanh(jnp.dot(z1, wp1_ref[...], preferred_element_type=jnp.float32)
                  + bp1_ref[...])
    w0 = jnp.sum(t0 * wp2_ref[...], axis=1, keepdims=True)
    w1 = jnp.sum(t1 * wp2_ref[...], axis=1, keepdims=True)
    w_ref[...] = jnp.concatenate([w0, w1], axis=1)


def _ka(op, dp, bb, wp1, bp1r, wp2r):
    blk = _BLK
    return pl.pallas_call(
        _ka_body,
        grid=(_GRID,),
        in_specs=[
            pl.BlockSpec((_NC, blk, _D), lambda i: (0, i, 0)),
            pl.BlockSpec((_NC, blk, 16), lambda i: (0, i, 0)),
            pl.BlockSpec((_NC, _D), lambda i: (0, 0)),
            pl.BlockSpec((_D, _HID), lambda i: (0, 0)),
            pl.BlockSpec((1, _HID), lambda i: (0, 0)),
            pl.BlockSpec((1, _HID), lambda i: (0, 0)),
        ],
        out_specs=[
            pl.BlockSpec((blk, _D), lambda i: (i, 0)),
            pl.BlockSpec((blk, _D), lambda i: (i, 0)),
            pl.BlockSpec((blk, 2), lambda i: (i, 0)),
        ],
        out_shape=[
            jax.ShapeDtypeStruct((_N, _D), jnp.float32),
            jax.ShapeDtypeStruct((_N, _D), jnp.float32),
            jax.ShapeDtypeStruct((_N, 2), jnp.float32),
        ],
    )(op, dp, bb, wp1, bp1r, wp2r)


def _kb_body(z0_ref, z1_ref, w_ref, out_ref):
    w = w_ref[...]
    s0 = jnp.sum(w[:, 0:1]) / _N
    s1 = jnp.sum(w[:, 1:2]) / _N
    m = jnp.maximum(s0, s1)
    e0 = jnp.exp(s0 - m)
    e1 = jnp.exp(s1 - m)
    beta0 = e0 / (e0 + e1)
    beta1 = e1 / (e0 + e1)
    out_ref[...] = beta0 * z0_ref[...] + beta1 * z1_ref[...]


def _kb(z0, z1, w):
    blk = _BLK
    return pl.pallas_call(
        _kb_body,
        grid=(_GRID,),
        in_specs=[
            pl.BlockSpec((blk, _D), lambda i: (i, 0)),
            pl.BlockSpec((blk, _D), lambda i: (i, 0)),
            pl.BlockSpec((_N, 2), lambda i: (0, 0)),
        ],
        out_specs=pl.BlockSpec((blk, _D), lambda i: (i, 0)),
        out_shape=jax.ShapeDtypeStruct((_N, _D), jnp.float32),
    )(z0, z1, w)


def _attn_mats(attn_l, attn_r):
    k = jnp.arange(_D)
    hp = jnp.arange(16)
    mask = (k[:, None] // _OUT) == (hp[None, :] % _H)
    ml = jnp.where(mask, attn_l.reshape(_D)[:, None], 0.0)
    mr = jnp.where(mask, attn_r.reshape(_D)[:, None], 0.0)
    return ml.astype(jnp.float32), mr.astype(jnp.float32)


def kernel(x, edge_index_0, edge_index_1, W0, attn_l0, attn_r0, b0,
           W1, attn_l1, attn_r1, b1, Wp1, bp1, Wp2):
    src2 = jnp.stack([edge_index_0[0], edge_index_1[0]]).reshape(_NC * _R, _CHUNK)
    dst2 = jnp.stack([edge_index_0[1], edge_index_1[1]]).reshape(_NC * _R, _CHUNK)

    ml0, mr0 = _attn_mats(attn_l0, attn_r0)
    ml1, mr1 = _attn_mats(attn_l1, attn_r1)
    w = jnp.stack([W0, W1])
    ml = jnp.stack([ml0, ml1])
    mr = jnp.stack([mr0, mr1])
    bb = jnp.stack([b0, b1])

    feat, tl, tr, c = _pre(x, w, ml, mr)

    ee, dp = _p1(src2, dst2, tl.reshape(_NC * _N, 16),
                 tr.reshape(_NC * _N, 16), c.reshape(_NC, 16))
    op = _p2(src2, dst2, feat.reshape(_NC * _N, _D), ee)

    z0, z1, wsem = _ka(op, dp, bb, Wp1, bp1.reshape(1, _HID),
                       Wp2.reshape(1, _HID))
    return _kb(z0, z1, wsem)


# speedup vs baseline: 1.2632x; 1.0021x over previous
import jax
import jax.numpy as jnp
from jax import lax
from jax.experimental import pallas as pl
from jax.experimental.pallas import tpu as pltpu
from jax.experimental.pallas import tpu_sc as plsc

_N = 10000
_E = 320000
_IN = 128
_H = 8
_OUT = 16
_D = _H * _OUT
_HID = 128
_CHUNK = 128
_R = _E // _CHUNK
_NC = 2
_NS = 16
_SUB_BASE = 624
_SUB_CHUNKS = ((0, 128), (128, 128), (256, 128), (384, 128), (512, 112))
_TAIL_OFF = _SUB_BASE * _NS
_TAIL = _N - _TAIL_OFF
_LEAK = 0.2
_BLK = 2000
_GRID = _N // _BLK
_NB_BASE = _R // _NS
_NB_EXTRA = _R - _NB_BASE * _NS
_NPAIR = (_NB_BASE + _NB_EXTRA + 1) // 2


def _pre_body(x_ref, w_ref, ml_ref, mr_ref,
              feat_ref, tl_ref, tr_ref, c_ref, acc_ref):
    i = pl.program_id(1)
    x = x_ref[...]
    f = jnp.dot(x, w_ref[0], preferred_element_type=jnp.float32)
    feat_ref[0] = f
    tl = jnp.dot(f, ml_ref[0], preferred_element_type=jnp.float32)
    tr = jnp.dot(f, mr_ref[0], preferred_element_type=jnp.float32)
    tl_ref[0] = tl
    tr_ref[0] = tr
    for row, t in enumerate((tl, tr)):
        m = jnp.max(t, axis=0)
        prev = acc_ref[row, :]
        acc_ref[row, :] = jnp.where(i == 0, m, jnp.maximum(prev, m))
    zero = jnp.zeros((16,), jnp.float32)
    c_ref[0, 0, :] = jnp.maximum(zero, acc_ref[0, :] + acc_ref[1, :])


def _pre(x, w, ml, mr):
    blk = _BLK
    return pl.pallas_call(
        _pre_body,
        grid=(_NC, _GRID),
        in_specs=[
            pl.BlockSpec((blk, _IN), lambda p, i: (i, 0)),
            pl.BlockSpec((1, _IN, _D), lambda p, i: (p, 0, 0)),
            pl.BlockSpec((1, _D, 16), lambda p, i: (p, 0, 0)),
            pl.BlockSpec((1, _D, 16), lambda p, i: (p, 0, 0)),
        ],
        out_specs=[
            pl.BlockSpec((1, blk, _D), lambda p, i: (p, i, 0)),
            pl.BlockSpec((1, blk, 16), lambda p, i: (p, i, 0)),
            pl.BlockSpec((1, blk, 16), lambda p, i: (p, i, 0)),
            pl.BlockSpec((1, 1, 16), lambda p, i: (p, 0, 0)),
        ],
        out_shape=[
            jax.ShapeDtypeStruct((_NC, _N, _D), jnp.float32),
            jax.ShapeDtypeStruct((_NC, _N, 16), jnp.float32),
            jax.ShapeDtypeStruct((_NC, _N, 16), jnp.float32),
            jax.ShapeDtypeStruct((_NC, 1, 16), jnp.float32),
        ],
        scratch_shapes=[pltpu.VMEM((2, 16), jnp.float32)],
    )(x, w, ml, mr)


def _zero_shared(zbuf, sh, sid, width):
    base = pl.multiple_of(sid * _SUB_BASE, 8)
    for off, sz in _SUB_CHUNKS:
        pltpu.sync_copy(zbuf.at[pl.ds(0, sz)],
                        sh.at[pl.ds(pl.multiple_of(base + off, 8), sz)])

    @pl.when(sid == _NS - 1)
    def _zt():
        pltpu.sync_copy(zbuf.at[pl.ds(0, _TAIL)], sh.at[pl.ds(_TAIL_OFF, _TAIL)])


def _export_shared(sh, out2d_at_cid, sid):
    base = pl.multiple_of(sid * _SUB_BASE, 8)
    for off, sz in _SUB_CHUNKS:
        o = pl.multiple_of(base + off, 8)
        pltpu.sync_copy(sh.at[pl.ds(o, sz)], out2d_at_cid(o, sz))

    @pl.when(sid == _NS - 1)
    def _xt():
        pltpu.sync_copy(sh.at[pl.ds(_TAIL_OFF, _TAIL)],
                        out2d_at_cid(_TAIL_OFF, _TAIL))


def _p1_body(src_ref, dst_ref, tl_ref, tr_ref, c_ref,
             ee_ref, dp_ref,
             sxA, sxB, sxC, sxD, dxA, dxB, dxC, dxD, dgA, dgB, dgC, dgD,
             ab0, bb0, ab1, bb1, eb0, eb1, cbuf, dsh,
             siA, siB, siC, siD, sa0, sa1, sb0, sb1, eo0, eo1, sd0, sd1):
    cid = lax.axis_index("c")
    sid = lax.axis_index("s")
    sxs = (sxA, sxB, sxC, sxD)
    dxs = (dxA, dxB, dxC, dxD)
    dgs = (dgA, dgB, dgC, dgD)
    sis = (siA, siB, siC, siD)
    abs_ = (ab0, ab1)
    bbs = (bb0, bb1)
    ebs = (eb0, eb1)
    sas = (sa0, sa1)
    sbs = (sb0, sb1)
    eos = (eo0, eo1)
    sds = (sd0, sd1)

    def _z(i, _):
        eb0[i, :] = jnp.zeros((16,), jnp.float32)
        return 0
    lax.fori_loop(0, _CHUNK, _z, 0)
    _zero_shared(eb0, dsh, sid, 16)
    plsc.subcore_barrier()

    pltpu.sync_copy(c_ref, cbuf)
    cvec = cbuf[cid, :]
    noff = cid * _N

    nb = _NB_BASE + jnp.where(sid < _NB_EXTRA, 1, 0)

    def _pf(ci, k):
        r = cid * _R + sid + _NS * ci
        pltpu.async_copy(src_ref.at[r], sxs[k], sis[k])
        pltpu.async_copy(dst_ref.at[r], dxs[k], sis[k])

    def _launch(k, kf):
        pltpu.make_async_copy(src_ref.at[0], sxs[k], sis[k]).wait()
        pltpu.make_async_copy(src_ref.at[0], sxs[k], sis[k]).wait()
        sx = sxs[k]
        dx = dxs[k]
        dg = dgs[k]
        for kk in range(_CHUNK // 16):
            sx[pl.ds(16 * kk, 16)] = sx[pl.ds(16 * kk, 16)] + noff
            dg[pl.ds(16 * kk, 16)] = dx[pl.ds(16 * kk, 16)] + noff
        pltpu.async_copy(tl_ref.at[sx], abs_[kf], sas[kf])
        pltpu.async_copy(tr_ref.at[dg], bbs[kf], sbs[kf])

    def _comp(ci, k, kf):
        ab = abs_[kf]
        bb = bbs[kf]
        eb = ebs[kf]
        pltpu.make_async_copy(tl_ref.at[pl.ds(0, _CHUNK)], ab, sas[kf]).wait()
        pltpu.make_async_copy(tr_ref.at[pl.ds(0, _CHUNK)], bb, sbs[kf]).wait()

        @pl.when(ci >= 2)
        def _dr():
            pltpu.make_async_copy(eb, ee_ref.at[0], eos[kf]).wait()
            pltpu.make_async_copy(ab, dsh.at[pl.ds(0, _CHUNK)], sds[kf]).wait()

        def _edge(j, _):
            for u in range(4):
                jj = 4 * j + u
                e = ab[jj, :] + bb[jj, :]
                e = jnp.maximum(e, _LEAK * e) - cvec
                eb[jj, :] = jnp.exp(e)
            return 0
        lax.fori_loop(0, _CHUNK // 4, _edge, 0)
        r = cid * _R + sid + _NS * ci
        pltpu.async_copy(eb, ee_ref.at[r], eos[kf])
        pltpu.async_copy(eb, dsh.at[dxs[k]], sds[kf], add=True)

    _pf(0, 0)
    _pf(1, 1)
    _launch(0, 0)

    nsteps = (_NB_BASE + _NB_EXTRA + 3) // 4 + 1

    def _macro(i, _):
        for k in range(4):
            c = 4 * i + k

            @pl.when(c < nb)
            def _c():
                _comp(c, k % 4, k % 2)

            @pl.when(c + 1 < nb)
            def _l():
                _launch((k + 1) % 4, (k + 1) % 2)

            @pl.when(c + 2 < nb)
            def _p():
                _pf(c + 2, (k + 2) % 4)
        return 0
    lax.fori_loop(0, nsteps, _macro, 0)

    @pl.when(nb >= 1)
    def _dr0():
        pltpu.make_async_copy(eb0, ee_ref.at[0], eo0).wait()
        pltpu.make_async_copy(ab0, dsh.at[pl.ds(0, _CHUNK)], sd0).wait()

    @pl.when(nb >= 2)
    def _dr1():
        pltpu.make_async_copy(eb1, ee_ref.at[0], eo1).wait()
        pltpu.make_async_copy(ab1, dsh.at[pl.ds(0, _CHUNK)], sd1).wait()
    plsc.subcore_barrier()
    _export_shared(dsh, lambda o, sz: dp_ref.at[cid, pl.ds(o, sz)], sid)


def _p1(src2, dst2, tl, tr, c):
    mesh = plsc.VectorSubcoreMesh(core_axis_name="c", subcore_axis_name="s")
    f = pl.kernel(
        _p1_body,
        out_type=[
            jax.ShapeDtypeStruct((_NC * _R, _CHUNK, 16), jnp.float32),
            jax.ShapeDtypeStruct((_NC, _N, 16), jnp.float32),
        ],
        mesh=mesh,
        compiler_params=pltpu.CompilerParams(use_tc_tiling_on_sc=False),
        scratch_types=(
            [pltpu.VMEM((_CHUNK,), jnp.int32)] * 12
            + [pltpu.VMEM((_CHUNK, 16), jnp.float32)] * 6
            + [pltpu.VMEM((_NC, 16), jnp.float32)]
            + [pltpu.VMEM_SHARED((_N, 16), jnp.float32)]
            + [pltpu.SemaphoreType.DMA] * 12
        ),
    )
    return f(src2, dst2, tl, tr, c)


def _p2_body(src_ref, dst_ref, feat_ref, ee_ref,
             op_ref,
             sxA, sxB, sxC, dxA, dxB, dxC, ebA, ebB, ebC, fb0, fb1, osh,
             siA, siB, siC, seA, seB, seC, sf0, sf1, so0, so1):
    cid = lax.axis_index("c")
    sid = lax.axis_index("s")
    sxs = (sxA, sxB, sxC)
    dxs = (dxA, dxB, dxC)
    ebs = (ebA, ebB, ebC)
    sis = (siA, siB, siC)
    ses = (seA, seB, seC)
    fbs = (fb0, fb1)
    sfs = (sf0, sf1)
    sos = (so0, so1)

    def _z(i, _):
        for h in range(_H):
            fb0[i, pl.ds(16 * h, 16)] = jnp.zeros((16,), jnp.float32)
        return 0
    lax.fori_loop(0, _CHUNK, _z, 0)
    _zero_shared(fb0, osh, sid, _D)
    plsc.subcore_barrier()

    noff = cid * _N
    nb = _NB_BASE + jnp.where(sid < _NB_EXTRA, 1, 0)

    def _pf(ci, k):
        r = cid * _R + sid + _NS * ci
        pltpu.async_copy(src_ref.at[r], sxs[k], sis[k])
        pltpu.async_copy(dst_ref.at[r], dxs[k], sis[k])
        pltpu.async_copy(ee_ref.at[r], ebs[k], ses[k])

    def _launch(ci, k, kf):
        @pl.when(ci >= 2)
        def _dr():
            pltpu.make_async_copy(fbs[kf], osh.at[pl.ds(0, _CHUNK)],
                                  sos[kf]).wait()
        pltpu.make_async_copy(src_ref.at[0], sxs[k], sis[k]).wait()
        pltpu.make_async_copy(src_ref.at[0], sxs[k], sis[k]).wait()
        sx = sxs[k]
        for kk in range(_CHUNK // 16):
            sx[pl.ds(16 * kk, 16)] = sx[pl.ds(16 * kk, 16)] + noff
        pltpu.async_copy(feat_ref.at[sx], fbs[kf], sfs[kf])

    def _comp(k, kf):
        fb = fbs[kf]
        eb = ebs[k]
        pltpu.make_async_copy(feat_ref.at[pl.ds(0, _CHUNK)], fb, sfs[kf]).wait()
        pltpu.make_async_copy(ee_ref.at[0], eb, ses[k]).wait()

        def _edge(j, _):
            for u in range(4):
                jj = 4 * j + u
                a = eb[jj, :]
                for h in range(_H):
                    fb[jj, pl.ds(16 * h, 16)] = (
                        fb[jj, pl.ds(16 * h, 16)] * a[h])
            return 0
        lax.fori_loop(0, _CHUNK // 4, _edge, 0)
        pltpu.async_copy(fb, osh.at[dxs[k]], sos[kf], add=True)

    _pf(0, 0)
    _pf(1, 1)
    _launch(0, 0, 0)

    nsteps = (_NB_BASE + _NB_EXTRA + 5) // 6 + 1

    def _macro(i, _):
        for k in range(6):
            c = 6 * i + k

            @pl.when(c < nb)
            def _c():
                _comp(k % 3, k % 2)

            @pl.when(c + 1 < nb)
            def _l():
                _launch(c + 1, (k + 1) % 3, (k + 1) % 2)

            @pl.when(c + 2 < nb)
            def _p():
                _pf(c + 2, (k + 2) % 3)
        return 0
    lax.fori_loop(0, nsteps, _macro, 0)

    @pl.when(nb >= 1)
    def _dr0():
        pltpu.make_async_copy(fb0, osh.at[pl.ds(0, _CHUNK)], so0).wait()

    @pl.when(nb >= 2)
    def _dr1():
        pltpu.make_async_copy(fb1, osh.at[pl.ds(0, _CHUNK)], so1).wait()
    plsc.subcore_barrier()
    _export_shared(osh, lambda o, sz: op_ref.at[cid, pl.ds(o, sz)], sid)


def _p2(src2, dst2, feat, ee):
    mesh = plsc.VectorSubcoreMesh(core_axis_name="c", subcore_axis_name="s")
    f = pl.kernel(
        _p2_body,
        out_type=jax.ShapeDtypeStruct((_NC, _N, _D), jnp.float32),
        mesh=mesh,
        compiler_params=pltpu.CompilerParams(use_tc_tiling_on_sc=False),
        scratch_types=(
            [pltpu.VMEM((_CHUNK,), jnp.int32)] * 6
            + [pltpu.VMEM((_CHUNK, 16), jnp.float32)] * 3
            + [pltpu.VMEM((_CHUNK, _D), jnp.float32)] * 2
            + [pltpu.VMEM_SHARED((_N, _D), jnp.float32)]
            + [pltpu.SemaphoreType.DMA] * 10
        ),
    )
    return f(src2, dst2, feat, ee)


def _ka_body(op_ref, dp_ref, bb_ref, wp1_ref, bp1_ref, wp2_ref,
             z0_ref, z1_ref, w_ref):
    lane = lax.broadcasted_iota(jnp.int32, (8, _D), 1)
    row = lax.broadcasted_iota(jnp.int32, (8, _D), 0)
    eh = jnp.where(lane // _OUT == row, 1.0, 0.0)

    r0 = 1.0 / (dp_ref[0][:, 0:8] + 1e-30)
    r1 = 1.0 / (dp_ref[1][:, 0:8] + 1e-30)
    s0 = jnp.dot(r0, eh, preferred_element_type=jnp.float32)
    s1 = jnp.dot(r1, eh, preferred_element_type=jnp.float32)
    z0 = op_ref[0] * s0 + bb_ref[0:1, :]
    z1 = op_ref[1] * s1 + bb_ref[1:2, :]
    z0_ref[...] = z0
    z1_ref[...] = z1
    t0 = jnp.tanh(jnp.dot(z0, wp1_ref[...], preferred_element_type=jnp.float32)
                  + bp1_ref[...])
    t1 = jnp.tanh(jnp.dot(z1, wp1_ref[...], preferred_element_type=jnp.float32)
                  + bp1_ref[...])
    w0 = jnp.sum(t0 * wp2_ref[...], axis=1, keepdims=True)
    w1 = jnp.sum(t1 * wp2_ref[...], axis=1, keepdims=True)
    w_ref[...] = jnp.concatenate([w0, w1], axis=1)


def _ka(op, dp, bb, wp1, bp1r, wp2r):
    blk = _BLK
    return pl.pallas_call(
        _ka_body,
        grid=(_GRID,),
        in_specs=[
            pl.BlockSpec((_NC, blk, _D), lambda i: (0, i, 0)),
            pl.BlockSpec((_NC, blk, 16), lambda i: (0, i, 0)),
            pl.BlockSpec((_NC, _D), lambda i: (0, 0)),
            pl.BlockSpec((_D, _HID), lambda i: (0, 0)),
            pl.BlockSpec((1, _HID), lambda i: (0, 0)),
            pl.BlockSpec((1, _HID), lambda i: (0, 0)),
        ],
        out_specs=[
            pl.BlockSpec((blk, _D), lambda i: (i, 0)),
            pl.BlockSpec((blk, _D), lambda i: (i, 0)),
            pl.BlockSpec((blk, 2), lambda i: (i, 0)),
        ],
        out_shape=[
            jax.ShapeDtypeStruct((_N, _D), jnp.float32),
            jax.ShapeDtypeStruct((_N, _D), jnp.float32),
            jax.ShapeDtypeStruct((_N, 2), jnp.float32),
        ],
    )(op, dp, bb, wp1, bp1r, wp2r)


def _kb_body(z0_ref, z1_ref, w_ref, out_ref):
    w = w_ref[...]
    s0 = jnp.sum(w[:, 0:1]) / _N
    s1 = jnp.sum(w[:, 1:2]) / _N
    m = jnp.maximum(s0, s1)
    e0 = jnp.exp(s0 - m)
    e1 = jnp.exp(s1 - m)
    beta0 = e0 / (e0 + e1)
    beta1 = e1 / (e0 + e1)
    out_ref[...] = beta0 * z0_ref[...] + beta1 * z1_ref[...]


def _kb(z0, z1, w):
    blk = _BLK
    return pl.pallas_call(
        _kb_body,
        grid=(_GRID,),
        in_specs=[
            pl.BlockSpec((blk, _D), lambda i: (i, 0)),
            pl.BlockSpec((blk, _D), lambda i: (i, 0)),
            pl.BlockSpec((_N, 2), lambda i: (0, 0)),
        ],
        out_specs=pl.BlockSpec((blk, _D), lambda i: (i, 0)),
        out_shape=jax.ShapeDtypeStruct((_N, _D), jnp.float32),
    )(z0, z1, w)


def _attn_mats(attn_l, attn_r):
    k = jnp.arange(_D)
    hp = jnp.arange(16)
    mask = (k[:, None] // _OUT) == (hp[None, :] % _H)
    ml = jnp.where(mask, attn_l.reshape(_D)[:, None], 0.0)
    mr = jnp.where(mask, attn_r.reshape(_D)[:, None], 0.0)
    return ml.astype(jnp.float32), mr.astype(jnp.float32)


def kernel(x, edge_index_0, edge_index_1, W0, attn_l0, attn_r0, b0,
           W1, attn_l1, attn_r1, b1, Wp1, bp1, Wp2):
    src2 = jnp.stack([edge_index_0[0], edge_index_1[0]]).reshape(_NC * _R, _CHUNK)
    dst2 = jnp.stack([edge_index_0[1], edge_index_1[1]]).reshape(_NC * _R, _CHUNK)

    ml0, mr0 = _attn_mats(attn_l0, attn_r0)
    ml1, mr1 = _attn_mats(attn_l1, attn_r1)
    w = jnp.stack([W0, W1])
    ml = jnp.stack([ml0, ml1])
    mr = jnp.stack([mr0, mr1])
    bb = jnp.stack([b0, b1])

    feat, tl, tr, c = _pre(x, w, ml, mr)

    ee, dp = _p1(src2, dst2, tl.reshape(_NC * _N, 16),
                 tr.reshape(_NC * _N, 16), c.reshape(_NC, 16))
    op = _p2(src2, dst2, feat.reshape(_NC * _N, _D), ee)

    z0, z1, wsem = _ka(op, dp, bb, Wp1, bp1.reshape(1, _HID),
                       Wp2.reshape(1, _HID))
    return _kb(z0, z1, wsem)
